# Initial kernel scaffold; baseline (speedup 1.0000x reference)
import jax
import jax.numpy as jnp
from jax.experimental import pallas as pl


def kernel(x, edge_index, edge_attr, Wl0, bl0, Wr0, We0, be0, Wl1, bl1, Wr1, We1, be1):
    raise NotImplementedError("write your pallas kernel here")


# speedup vs baseline: 4.0487x; 4.0487x over previous
import functools

import jax
import jax.numpy as jnp
from jax import lax
from jax.experimental import pallas as pl
from jax.experimental.pallas import tpu as pltpu
from jax.experimental.pallas import tpu_sc as plsc

N = 10000
E = 320000
D = 128
ED = 16

NC = 2
NS = 16
NW = NC * NS
EPW = E // NW
C = 80
NCHUNK = EPW // C
NP = 10240
RPT = NP // NS

_MESH = plsc.VectorSubcoreMesh(core_axis_name="c", subcore_axis_name="s")


def _edge_agg(dst, ea, z128, init128):
    @functools.partial(
        pl.kernel,
        out_type=jax.ShapeDtypeStruct((NC, NP, D), jnp.float32),
        mesh=_MESH,
        scratch_types=[
            pltpu.VMEM_SHARED((NP, D), jnp.float32),
            pltpu.VMEM((C,), jnp.int32),
            pltpu.VMEM((C, ED), jnp.float32),
            pltpu.VMEM((C, D), jnp.float32),
        ],
    )
    def k(dst_hbm, ea_hbm, z128_hbm, init128_hbm,
          outEA,
          EA_sh, dst_v, ea_v, scat_v):
        c = lax.axis_index("c")
        s = lax.axis_index("s")
        r0 = s * RPT
        pltpu.sync_copy(z128_hbm, EA_sh.at[pl.ds(r0, RPT)])
        pltpu.sync_copy(init128_hbm, scat_v)
        plsc.subcore_barrier()

        ebase = (c * NS + s) * EPW

        def step(i, carry):
            b = ebase + i * C
            pltpu.sync_copy(dst_hbm.at[pl.ds(b, C)], dst_v)
            pltpu.sync_copy(ea_hbm.at[pl.ds(b, C)], ea_v)
            for e in range(C):
                scat_v[e, 0:ED] = ea_v[e, :]
            pltpu.sync_copy(scat_v, EA_sh.at[dst_v], add=True)
            return carry

        lax.fori_loop(0, NCHUNK, step, 0)
        plsc.subcore_barrier()
        pltpu.sync_copy(EA_sh.at[pl.ds(r0, RPT)], outEA.at[c, pl.ds(r0, RPT)])

    return k(dst, ea, z128, init128)


def _spmm(h, src, dst, z128):
    @functools.partial(
        pl.kernel,
        out_type=jax.ShapeDtypeStruct((NC, NP, D), jnp.float32),
        mesh=_MESH,
        scratch_types=[
            pltpu.VMEM_SHARED((NP, D), jnp.float32),
            pltpu.VMEM((C,), jnp.int32),
            pltpu.VMEM((C,), jnp.int32),
            pltpu.VMEM((C, D), jnp.float32),
            pltpu.SemaphoreType.DMA,
        ],
    )
    def k(h_hbm, src_hbm, dst_hbm, z128_hbm, outS,
          S_sh, src_v, dst_v, rows_v, sem):
        c = lax.axis_index("c")
        s = lax.axis_index("s")
        r0 = s * RPT
        pltpu.sync_copy(z128_hbm, S_sh.at[pl.ds(r0, RPT)])
        plsc.subcore_barrier()

        ebase = (c * NS + s) * EPW

        def step(i, carry):
            b = ebase + i * C
            pltpu.sync_copy(src_hbm.at[pl.ds(b, C)], src_v)
            pltpu.sync_copy(dst_hbm.at[pl.ds(b, C)], dst_v)
            pltpu.async_copy(h_hbm.at[src_v], rows_v, sem).wait()
            pltpu.sync_copy(rows_v, S_sh.at[dst_v], add=True)
            return carry

        lax.fori_loop(0, NCHUNK, step, 0)
        plsc.subcore_barrier()
        pltpu.sync_copy(S_sh.at[pl.ds(r0, RPT)], outS.at[c, pl.ds(r0, RPT)])

    return k(h, src, dst, z128)


_BLK = 1000


def _dense_body(Sp_ref, EAp_ref, x_ref, WeT_ref, be_ref,
                WlT_ref, bl_ref, WrT_ref, o_ref):
    s = Sp_ref[0] + Sp_ref[1]
    eac = EAp_ref[0] + EAp_ref[1]
    ea = eac[:, 0:ED]
    cnt = eac[:, ED:ED + 1]
    inv = 1.0 / jnp.maximum(cnt, 1.0)
    agg = (s + jnp.dot(ea, WeT_ref[...], preferred_element_type=jnp.float32)
           + cnt * be_ref[...]) * inv
    o_ref[...] = jnp.maximum(
        jnp.dot(agg, WlT_ref[...], preferred_element_type=jnp.float32)
        + bl_ref[...]
        + jnp.dot(x_ref[...], WrT_ref[...], preferred_element_type=jnp.float32),
        0.0)


def _dense(Sp, EAp, xin, WeT, be, WlT, bl, WrT):
    return pl.pallas_call(
        _dense_body,
        grid=(N // _BLK,),
        in_specs=[
            pl.BlockSpec((NC, _BLK, D), lambda i: (0, i, 0)),
            pl.BlockSpec((NC, _BLK, D), lambda i: (0, i, 0)),
            pl.BlockSpec((_BLK, D), lambda i: (i, 0)),
            pl.BlockSpec((ED, D), lambda i: (0, 0)),
            pl.BlockSpec((1, D), lambda i: (0, 0)),
            pl.BlockSpec((D, D), lambda i: (0, 0)),
            pl.BlockSpec((1, D), lambda i: (0, 0)),
            pl.BlockSpec((D, D), lambda i: (0, 0)),
        ],
        out_specs=pl.BlockSpec((_BLK, D), lambda i: (i, 0)),
        out_shape=jax.ShapeDtypeStruct((N, D), jnp.float32),
    )(Sp, EAp, xin, WeT, be, WlT, bl, WrT)


def kernel(x, edge_index, edge_attr, Wl0, bl0, Wr0, We0, be0,
           Wl1, bl1, Wr1, We1, be1):
    src = edge_index[0]
    dst = edge_index[1]
    z128 = jnp.zeros((RPT, D), jnp.float32)
    init128 = jnp.zeros((C, D), jnp.float32).at[:, ED].set(1.0)

    EAp = _edge_agg(dst, edge_attr, z128, init128)
    S0p = _spmm(x, src, dst, z128)
    h = _dense(S0p, EAp, x, We0.T, be0.reshape(1, D), Wl0.T,
               bl0.reshape(1, D), Wr0.T)
    S1p = _spmm(h, src, dst, z128)
    out = _dense(S1p, EAp, h, We1.T, be1.reshape(1, D), Wl1.T,
                 bl1.reshape(1, D), Wr1.T)
    return out


# speedup vs baseline: 5.4903x; 1.3561x over previous
import functools

import jax
import jax.numpy as jnp
from jax import lax
from jax.experimental import pallas as pl
from jax.experimental.pallas import tpu as pltpu
from jax.experimental.pallas import tpu_sc as plsc

N = 10000
E = 320000
D = 128
ED = 16

NC = 2
NS = 16
NW = NC * NS
EPW = E // NW
C = 80
NCHUNK = EPW // C
NP = 10240
RPT = NP // NS

_MESH = plsc.VectorSubcoreMesh(core_axis_name="c", subcore_axis_name="s")


def _edge_agg(dst, ea, z128, init128):
    @functools.partial(
        pl.kernel,
        out_type=jax.ShapeDtypeStruct((NC, NP, D), jnp.float32),
        mesh=_MESH,
        scratch_types=[
            pltpu.VMEM_SHARED((NP, D), jnp.float32),
            pltpu.VMEM((C,), jnp.int32),
            pltpu.VMEM((C, ED), jnp.float32),
            pltpu.VMEM((C, D), jnp.float32),
        ],
    )
    def k(dst_hbm, ea_hbm, z128_hbm, init128_hbm,
          outEA,
          EA_sh, dst_v, ea_v, scat_v):
        c = lax.axis_index("c")
        s = lax.axis_index("s")
        r0 = s * RPT
        pltpu.sync_copy(z128_hbm, EA_sh.at[pl.ds(r0, RPT)])
        pltpu.sync_copy(init128_hbm, scat_v)
        plsc.subcore_barrier()

        ebase = (c * NS + s) * EPW

        def step(i, carry):
            b = ebase + i * C
            pltpu.sync_copy(dst_hbm.at[pl.ds(b, C)], dst_v)
            pltpu.sync_copy(ea_hbm.at[pl.ds(b, C)], ea_v)
            for e in range(C):
                scat_v[e, 0:ED] = ea_v[e, :]
            pltpu.sync_copy(scat_v, EA_sh.at[dst_v], add=True)
            return carry

        lax.fori_loop(0, NCHUNK, step, 0)
        plsc.subcore_barrier()
        pltpu.sync_copy(EA_sh.at[pl.ds(r0, RPT)], outEA.at[c, pl.ds(r0, RPT)])

    return k(dst, ea, z128, init128)


CS = 50
KS = 2
GS = EPW // (CS * KS)


def _spmm(h, src3, dst3, z128):
    @functools.partial(
        pl.kernel,
        out_type=jax.ShapeDtypeStruct((NC, NP, D), jnp.float32),
        mesh=_MESH,
        scratch_types=[
            pltpu.VMEM_SHARED((NP, D), jnp.float32),
            pltpu.VMEM((KS, CS), jnp.int32),
            pltpu.VMEM((KS, CS), jnp.int32),
            pltpu.VMEM((KS, CS), jnp.int32),
            pltpu.VMEM((KS, CS), jnp.int32),
            pltpu.VMEM((KS, CS, D), jnp.float32),
            pltpu.VMEM((KS, CS, D), jnp.float32),
            pltpu.SemaphoreType.DMA,
            pltpu.SemaphoreType.DMA,
            pltpu.SemaphoreType.DMA,
            pltpu.SemaphoreType.DMA,
        ],
    )
    def k(h_hbm, src3_hbm, dst3_hbm, z128_hbm, outS,
          S_sh, srcb0, srcb1, dstb0, dstb1, rows0, rows1,
          gsem0, gsem1, ssem0, ssem1):
        c = lax.axis_index("c")
        s = lax.axis_index("s")
        r0 = s * RPT
        pltpu.sync_copy(z128_hbm, S_sh.at[pl.ds(r0, RPT)])
        plsc.subcore_barrier()

        srcb = (srcb0, srcb1)
        dstb = (dstb0, dstb1)
        rows = (rows0, rows1)
        gsem = (gsem0, gsem1)
        ssem = (ssem0, ssem1)
        gbase = (c * NS + s) * GS

        def fire(g, hf):
            row = gbase + g
            pltpu.sync_copy(src3_hbm.at[row], srcb[hf])
            pltpu.sync_copy(dst3_hbm.at[row], dstb[hf])
            for kk in range(KS):
                pltpu.async_copy(h_hbm.at[srcb[hf].at[kk]], rows[hf].at[kk],
                                 gsem[hf])

        def drain_g(hf):
            for kk in range(KS):
                pltpu.make_async_copy(h_hbm.at[srcb[hf].at[kk]],
                                      rows[hf].at[kk], gsem[hf]).wait()

        def fire_s(hf):
            for kk in range(KS):
                pltpu.async_copy(rows[hf].at[kk],
                                 S_sh.at[dstb[hf].at[kk]], ssem[hf], add=True)

        def drain_s(hf):
            for kk in range(KS):
                pltpu.make_async_copy(rows[hf].at[kk],
                                      S_sh.at[dstb[hf].at[kk]],
                                      ssem[hf]).wait()

        fire(0, 0)
        fire(1, 1)
        drain_g(0)
        fire_s(0)
        drain_s(0)
        fire(2, 0)
        drain_g(1)
        fire_s(1)

        def body(t, carry):
            drain_s(1)
            fire(2 * t + 1, 1)
            drain_g(0)
            fire_s(0)
            drain_s(0)
            fire(2 * t + 2, 0)
            drain_g(1)
            fire_s(1)
            return carry

        lax.fori_loop(1, GS // 2 - 1, body, 0)

        drain_s(1)
        fire(GS - 1, 1)
        drain_g(0)
        fire_s(0)
        drain_s(0)
        drain_g(1)
        fire_s(1)
        drain_s(1)

        plsc.subcore_barrier()
        pltpu.sync_copy(S_sh.at[pl.ds(r0, RPT)], outS.at[c, pl.ds(r0, RPT)])

    return k(h, src3, dst3, z128)


_BLK = 1000


def _dense_body(Sp_ref, EAp_ref, x_ref, WeT_ref, be_ref,
                WlT_ref, bl_ref, WrT_ref, o_ref):
    s = Sp_ref[0] + Sp_ref[1]
    eac = EAp_ref[0] + EAp_ref[1]
    ea = eac[:, 0:ED]
    cnt = eac[:, ED:ED + 1]
    inv = 1.0 / jnp.maximum(cnt, 1.0)
    agg = (s + jnp.dot(ea, WeT_ref[...], preferred_element_type=jnp.float32)
           + cnt * be_ref[...]) * inv
    o_ref[...] = jnp.maximum(
        jnp.dot(agg, WlT_ref[...], preferred_element_type=jnp.float32)
        + bl_ref[...]
        + jnp.dot(x_ref[...], WrT_ref[...], preferred_element_type=jnp.float32),
        0.0)


def _dense(Sp, EAp, xin, WeT, be, WlT, bl, WrT):
    return pl.pallas_call(
        _dense_body,
        grid=(N // _BLK,),
        in_specs=[
            pl.BlockSpec((NC, _BLK, D), lambda i: (0, i, 0)),
            pl.BlockSpec((NC, _BLK, D), lambda i: (0, i, 0)),
            pl.BlockSpec((_BLK, D), lambda i: (i, 0)),
            pl.BlockSpec((ED, D), lambda i: (0, 0)),
            pl.BlockSpec((1, D), lambda i: (0, 0)),
            pl.BlockSpec((D, D), lambda i: (0, 0)),
            pl.BlockSpec((1, D), lambda i: (0, 0)),
            pl.BlockSpec((D, D), lambda i: (0, 0)),
        ],
        out_specs=pl.BlockSpec((_BLK, D), lambda i: (i, 0)),
        out_shape=jax.ShapeDtypeStruct((N, D), jnp.float32),
    )(Sp, EAp, xin, WeT, be, WlT, bl, WrT)


def kernel(x, edge_index, edge_attr, Wl0, bl0, Wr0, We0, be0,
           Wl1, bl1, Wr1, We1, be1):
    src = edge_index[0]
    dst = edge_index[1]
    z128 = jnp.zeros((RPT, D), jnp.float32)
    init128 = jnp.zeros((C, D), jnp.float32).at[:, ED].set(1.0)

    src3 = src.reshape(E // (KS * CS), KS, CS)
    dst3 = dst.reshape(E // (KS * CS), KS, CS)

    EAp = _edge_agg(dst, edge_attr, z128, init128)
    S0p = _spmm(x, src3, dst3, z128)
    h = _dense(S0p, EAp, x, We0.T, be0.reshape(1, D), Wl0.T,
               bl0.reshape(1, D), Wr0.T)
    S1p = _spmm(h, src3, dst3, z128)
    out = _dense(S1p, EAp, h, We1.T, be1.reshape(1, D), Wl1.T,
                 bl1.reshape(1, D), Wr1.T)
    return out


# speedup vs baseline: 6.8433x; 1.2464x over previous
import functools

import jax
import jax.numpy as jnp
from jax import lax
from jax.experimental import pallas as pl
from jax.experimental.pallas import tpu as pltpu
from jax.experimental.pallas import tpu_sc as plsc

N = 10000
E = 320000
D = 128
ED = 16

NC = 2
NS = 16
NW = NC * NS
EPW = E // NW
C = 80
NCHUNK = EPW // C
NP = 10240
RPT = NP // NS

_MESH = plsc.VectorSubcoreMesh(core_axis_name="c", subcore_axis_name="s")


def _edge_agg(dst3e, ea, z128, init128):
    @functools.partial(
        pl.kernel,
        out_type=jax.ShapeDtypeStruct((NC, NP, D), jnp.float32),
        mesh=_MESH,
        scratch_types=[
            pltpu.VMEM_SHARED((NP, D), jnp.float32),
            pltpu.VMEM((1, C), jnp.int32),
            pltpu.VMEM((1, C), jnp.int32),
            pltpu.VMEM((C, ED), jnp.float32),
            pltpu.VMEM((C, ED), jnp.float32),
            pltpu.VMEM((C, D), jnp.float32),
            pltpu.VMEM((C, D), jnp.float32),
            pltpu.SemaphoreType.DMA,
            pltpu.SemaphoreType.DMA,
            pltpu.SemaphoreType.DMA,
            pltpu.SemaphoreType.DMA,
        ],
    )
    def k(dst3e_hbm, ea_hbm, z128_hbm, init128_hbm,
          outEA,
          EA_sh, dstb0, dstb1, eab0, eab1, scat0, scat1,
          lsem0, lsem1, ssem0, ssem1):
        c = lax.axis_index("c")
        s = lax.axis_index("s")
        r0 = s * RPT
        pltpu.sync_copy(z128_hbm, EA_sh.at[pl.ds(r0, RPT)])
        pltpu.sync_copy(init128_hbm, scat0)
        pltpu.sync_copy(init128_hbm, scat1)
        plsc.subcore_barrier()

        dstb = (dstb0, dstb1)
        eab = (eab0, eab1)
        scat = (scat0, scat1)
        lsem = (lsem0, lsem1)
        ssem = (ssem0, ssem1)
        ebase = (c * NS + s) * NCHUNK

        def fire_loads(i, hf):
            row = ebase + i
            pltpu.async_copy(ea_hbm.at[pl.ds(row * C, C)], eab[hf], lsem[hf])
            pltpu.async_copy(dst3e_hbm.at[row], dstb[hf], lsem[hf])

        def drain_loads(i, hf):
            row = ebase + i
            pltpu.make_async_copy(ea_hbm.at[pl.ds(row * C, C)], eab[hf],
                                  lsem[hf]).wait()
            pltpu.make_async_copy(dst3e_hbm.at[row], dstb[hf], lsem[hf]).wait()

        def assemble(hf):
            for e in range(C):
                scat[hf][e, 0:ED] = eab[hf][e, :]

        def fire_s(hf):
            pltpu.async_copy(scat[hf], EA_sh.at[dstb[hf].at[0]], ssem[hf],
                             add=True)

        def drain_s(hf):
            pltpu.make_async_copy(scat[hf], EA_sh.at[dstb[hf].at[0]],
                                  ssem[hf]).wait()

        fire_loads(0, 0)
        fire_loads(1, 1)
        drain_loads(0, 0)
        assemble(0)
        fire_s(0)

        def body(t, carry):
            i1 = 2 * t + 1
            drain_s(0)
            fire_loads(i1 + 1, 0)
            drain_loads(i1, 1)
            assemble(1)
            fire_s(1)
            drain_s(1)
            fire_loads(i1 + 2, 1)
            drain_loads(i1 + 1, 0)
            assemble(0)
            fire_s(0)
            return carry

        lax.fori_loop(0, (NCHUNK - 3) // 2, body, 0)

        drain_s(0)
        fire_loads(NCHUNK - 1, 0)
        drain_loads(NCHUNK - 2, 1)
        assemble(1)
        fire_s(1)
        drain_s(1)
        drain_loads(NCHUNK - 1, 0)
        assemble(0)
        fire_s(0)
        drain_s(0)

        plsc.subcore_barrier()
        pltpu.sync_copy(EA_sh.at[pl.ds(r0, RPT)], outEA.at[c, pl.ds(r0, RPT)])

    return k(dst3e, ea, z128, init128)


CS = 50
KS = 2
GS = EPW // (CS * KS)


def _spmm(h, src3, dst3, z128):
    @functools.partial(
        pl.kernel,
        out_type=jax.ShapeDtypeStruct((NC, NP, D), jnp.float32),
        mesh=_MESH,
        scratch_types=[
            pltpu.VMEM_SHARED((NP, D), jnp.float32),
            pltpu.VMEM((KS, CS), jnp.int32),
            pltpu.VMEM((KS, CS), jnp.int32),
            pltpu.VMEM((KS, CS), jnp.int32),
            pltpu.VMEM((KS, CS), jnp.int32),
            pltpu.VMEM((KS, CS, D), jnp.float32),
            pltpu.VMEM((KS, CS, D), jnp.float32),
            pltpu.SemaphoreType.DMA,
            pltpu.SemaphoreType.DMA,
            pltpu.SemaphoreType.DMA,
            pltpu.SemaphoreType.DMA,
        ],
    )
    def k(h_hbm, src3_hbm, dst3_hbm, z128_hbm, outS,
          S_sh, srcb0, srcb1, dstb0, dstb1, rows0, rows1,
          gsem0, gsem1, ssem0, ssem1):
        c = lax.axis_index("c")
        s = lax.axis_index("s")
        r0 = s * RPT
        pltpu.sync_copy(z128_hbm, S_sh.at[pl.ds(r0, RPT)])
        plsc.subcore_barrier()

        srcb = (srcb0, srcb1)
        dstb = (dstb0, dstb1)
        rows = (rows0, rows1)
        gsem = (gsem0, gsem1)
        ssem = (ssem0, ssem1)
        gbase = (c * NS + s) * GS

        def fire(g, hf):
            row = gbase + g
            pltpu.sync_copy(src3_hbm.at[row], srcb[hf])
            pltpu.sync_copy(dst3_hbm.at[row], dstb[hf])
            for kk in range(KS):
                pltpu.async_copy(h_hbm.at[srcb[hf].at[kk]], rows[hf].at[kk],
                                 gsem[hf])

        def drain_g(hf):
            for kk in range(KS):
                pltpu.make_async_copy(h_hbm.at[srcb[hf].at[kk]],
                                      rows[hf].at[kk], gsem[hf]).wait()

        def fire_s(hf):
            for kk in range(KS):
                pltpu.async_copy(rows[hf].at[kk],
                                 S_sh.at[dstb[hf].at[kk]], ssem[hf], add=True)

        def drain_s(hf):
            for kk in range(KS):
                pltpu.make_async_copy(rows[hf].at[kk],
                                      S_sh.at[dstb[hf].at[kk]],
                                      ssem[hf]).wait()

        fire(0, 0)
        fire(1, 1)
        drain_g(0)
        fire_s(0)
        drain_s(0)
        fire(2, 0)
        drain_g(1)
        fire_s(1)

        def body(t, carry):
            drain_s(1)
            fire(2 * t + 1, 1)
            drain_g(0)
            fire_s(0)
            drain_s(0)
            fire(2 * t + 2, 0)
            drain_g(1)
            fire_s(1)
            return carry

        lax.fori_loop(1, GS // 2 - 1, body, 0)

        drain_s(1)
        fire(GS - 1, 1)
        drain_g(0)
        fire_s(0)
        drain_s(0)
        drain_g(1)
        fire_s(1)
        drain_s(1)

        plsc.subcore_barrier()
        pltpu.sync_copy(S_sh.at[pl.ds(r0, RPT)], outS.at[c, pl.ds(r0, RPT)])

    return k(h, src3, dst3, z128)


_BLK = 1000


def _dense_body(Sp_ref, EAp_ref, x_ref, WeT_ref, be_ref,
                WlT_ref, bl_ref, WrT_ref, o_ref):
    s = Sp_ref[0] + Sp_ref[1]
    eac = EAp_ref[0] + EAp_ref[1]
    ea = eac[:, 0:ED]
    cnt = eac[:, ED:ED + 1]
    inv = 1.0 / jnp.maximum(cnt, 1.0)
    agg = (s + jnp.dot(ea, WeT_ref[...], preferred_element_type=jnp.float32)
           + cnt * be_ref[...]) * inv
    o_ref[...] = jnp.maximum(
        jnp.dot(agg, WlT_ref[...], preferred_element_type=jnp.float32)
        + bl_ref[...]
        + jnp.dot(x_ref[...], WrT_ref[...], preferred_element_type=jnp.float32),
        0.0)


def _dense(Sp, EAp, xin, WeT, be, WlT, bl, WrT):
    return pl.pallas_call(
        _dense_body,
        grid=(N // _BLK,),
        in_specs=[
            pl.BlockSpec((NC, _BLK, D), lambda i: (0, i, 0)),
            pl.BlockSpec((NC, _BLK, D), lambda i: (0, i, 0)),
            pl.BlockSpec((_BLK, D), lambda i: (i, 0)),
            pl.BlockSpec((ED, D), lambda i: (0, 0)),
            pl.BlockSpec((1, D), lambda i: (0, 0)),
            pl.BlockSpec((D, D), lambda i: (0, 0)),
            pl.BlockSpec((1, D), lambda i: (0, 0)),
            pl.BlockSpec((D, D), lambda i: (0, 0)),
        ],
        out_specs=pl.BlockSpec((_BLK, D), lambda i: (i, 0)),
        out_shape=jax.ShapeDtypeStruct((N, D), jnp.float32),
    )(Sp, EAp, xin, WeT, be, WlT, bl, WrT)


def kernel(x, edge_index, edge_attr, Wl0, bl0, Wr0, We0, be0,
           Wl1, bl1, Wr1, We1, be1):
    src = edge_index[0]
    dst = edge_index[1]
    z128 = jnp.zeros((RPT, D), jnp.float32)
    init128 = jnp.zeros((C, D), jnp.float32).at[:, ED].set(1.0)

    src3 = src.reshape(E // (KS * CS), KS, CS)
    dst3 = dst.reshape(E // (KS * CS), KS, CS)
    dst3e = dst.reshape(E // C, 1, C)

    EAp = _edge_agg(dst3e, edge_attr, z128, init128)
    S0p = _spmm(x, src3, dst3, z128)
    h = _dense(S0p, EAp, x, We0.T, be0.reshape(1, D), Wl0.T,
               bl0.reshape(1, D), Wr0.T)
    S1p = _spmm(h, src3, dst3, z128)
    out = _dense(S1p, EAp, h, We1.T, be1.reshape(1, D), Wl1.T,
                 bl1.reshape(1, D), Wr1.T)
    return out


# speedup vs baseline: 8.7185x; 1.2740x over previous
import functools

import jax
import jax.numpy as jnp
from jax import lax
from jax.experimental import pallas as pl
from jax.experimental.pallas import tpu as pltpu
from jax.experimental.pallas import tpu_sc as plsc

N = 10000
E = 320000
D = 128
ED = 16

NC = 2
NS = 16
NW = NC * NS
EPW = E // NW
C = 80
NCHUNK = EPW // C
NP = 10240
RPT = NP // NS

_MESH = plsc.VectorSubcoreMesh(core_axis_name="c", subcore_axis_name="s")


def _edge_agg(dst3e, ea, z128, init128):
    @functools.partial(
        pl.kernel,
        out_type=jax.ShapeDtypeStruct((NC, NP, D), jnp.float32),
        mesh=_MESH,
        scratch_types=[
            pltpu.VMEM_SHARED((NP, D), jnp.float32),
            pltpu.VMEM((1, C), jnp.int32),
            pltpu.VMEM((1, C), jnp.int32),
            pltpu.VMEM((C, ED), jnp.float32),
            pltpu.VMEM((C, ED), jnp.float32),
            pltpu.VMEM((C, D), jnp.float32),
            pltpu.VMEM((C, D), jnp.float32),
            pltpu.SemaphoreType.DMA,
            pltpu.SemaphoreType.DMA,
            pltpu.SemaphoreType.DMA,
            pltpu.SemaphoreType.DMA,
        ],
    )
    def k(dst3e_hbm, ea_hbm, z128_hbm, init128_hbm,
          outEA,
          EA_sh, dstb0, dstb1, eab0, eab1, scat0, scat1,
          lsem0, lsem1, ssem0, ssem1):
        c = lax.axis_index("c")
        s = lax.axis_index("s")
        r0 = s * RPT
        pltpu.sync_copy(z128_hbm, EA_sh.at[pl.ds(r0, RPT)])
        pltpu.sync_copy(init128_hbm, scat0)
        pltpu.sync_copy(init128_hbm, scat1)
        plsc.subcore_barrier()

        dstb = (dstb0, dstb1)
        eab = (eab0, eab1)
        scat = (scat0, scat1)
        lsem = (lsem0, lsem1)
        ssem = (ssem0, ssem1)
        ebase = (c * NS + s) * NCHUNK

        def fire_loads(i, hf):
            row = ebase + i
            pltpu.async_copy(ea_hbm.at[pl.ds(row * C, C)], eab[hf], lsem[hf])
            pltpu.async_copy(dst3e_hbm.at[row], dstb[hf], lsem[hf])

        def drain_loads(i, hf):
            row = ebase + i
            pltpu.make_async_copy(ea_hbm.at[pl.ds(row * C, C)], eab[hf],
                                  lsem[hf]).wait()
            pltpu.make_async_copy(dst3e_hbm.at[row], dstb[hf], lsem[hf]).wait()

        def assemble(hf):
            for e in range(C):
                scat[hf][e, 0:ED] = eab[hf][e, :]

        def fire_s(hf):
            pltpu.async_copy(scat[hf], EA_sh.at[dstb[hf].at[0]], ssem[hf],
                             add=True)

        def drain_s(hf):
            pltpu.make_async_copy(scat[hf], EA_sh.at[dstb[hf].at[0]],
                                  ssem[hf]).wait()

        fire_loads(0, 0)
        fire_loads(1, 1)
        drain_loads(0, 0)
        assemble(0)
        fire_s(0)

        def body(t, carry):
            i1 = 2 * t + 1
            drain_s(0)
            fire_loads(i1 + 1, 0)
            drain_loads(i1, 1)
            assemble(1)
            fire_s(1)
            drain_s(1)
            fire_loads(i1 + 2, 1)
            drain_loads(i1 + 1, 0)
            assemble(0)
            fire_s(0)
            return carry

        lax.fori_loop(0, (NCHUNK - 3) // 2, body, 0)

        drain_s(0)
        fire_loads(NCHUNK - 1, 0)
        drain_loads(NCHUNK - 2, 1)
        assemble(1)
        fire_s(1)
        drain_s(1)
        drain_loads(NCHUNK - 1, 0)
        assemble(0)
        fire_s(0)
        drain_s(0)

        plsc.subcore_barrier()
        pltpu.sync_copy(EA_sh.at[pl.ds(r0, RPT)], outEA.at[c, pl.ds(r0, RPT)])

    return k(dst3e, ea, z128, init128)


CS = 125
GS = EPW // CS


def _spmm(h, src3, dst3, z128):
    @functools.partial(
        pl.kernel,
        out_type=jax.ShapeDtypeStruct((NC, NP, D), jnp.float32),
        mesh=_MESH,
        scratch_types=[
            pltpu.VMEM_SHARED((NP, D), jnp.float32),
            pltpu.VMEM((1, CS), jnp.int32),
            pltpu.VMEM((1, CS), jnp.int32),
            pltpu.VMEM((1, CS), jnp.int32),
            pltpu.VMEM((1, CS), jnp.int32),
            pltpu.VMEM((1, CS), jnp.int32),
            pltpu.VMEM((1, CS), jnp.int32),
            pltpu.VMEM((1, CS), jnp.int32),
            pltpu.VMEM((1, CS), jnp.int32),
            pltpu.VMEM((CS, D), jnp.float32),
            pltpu.VMEM((CS, D), jnp.float32),
            pltpu.SemaphoreType.DMA,
            pltpu.SemaphoreType.DMA,
            pltpu.SemaphoreType.DMA,
            pltpu.SemaphoreType.DMA,
            pltpu.SemaphoreType.DMA,
            pltpu.SemaphoreType.DMA,
            pltpu.SemaphoreType.DMA,
            pltpu.SemaphoreType.DMA,
        ],
    )
    def k(h_hbm, src3_hbm, dst3_hbm, z128_hbm, outS,
          S_sh, sb0, sb1, sb2, sb3, db0, db1, db2, db3, rows0, rows1,
          ls0, ls1, ls2, ls3, gsem0, gsem1, ssem0, ssem1):
        c = lax.axis_index("c")
        s = lax.axis_index("s")
        r0 = s * RPT
        pltpu.sync_copy(z128_hbm, S_sh.at[pl.ds(r0, RPT)])
        plsc.subcore_barrier()

        srcb = (sb0, sb1, sb2, sb3)
        dstb = (db0, db1, db2, db3)
        rows = (rows0, rows1)
        lsem = (ls0, ls1, ls2, ls3)
        gsem = (gsem0, gsem1)
        ssem = (ssem0, ssem1)
        gbase = (c * NS + s) * GS

        def fire_idx(g, slot):
            row = gbase + g
            pltpu.async_copy(src3_hbm.at[row], srcb[slot], lsem[slot])
            pltpu.async_copy(dst3_hbm.at[row], dstb[slot], lsem[slot])

        def fire_g(g, hf, slot):
            row = gbase + g
            pltpu.make_async_copy(src3_hbm.at[row], srcb[slot],
                                  lsem[slot]).wait()
            pltpu.make_async_copy(dst3_hbm.at[row], dstb[slot],
                                  lsem[slot]).wait()
            pltpu.async_copy(h_hbm.at[srcb[slot].at[0]], rows[hf], gsem[hf])

        def drain_g(hf):
            pltpu.make_async_copy(h_hbm.at[srcb[0].at[0]], rows[hf],
                                  gsem[hf]).wait()

        def fire_s(hf, slot):
            pltpu.async_copy(rows[hf], S_sh.at[dstb[slot].at[0]], ssem[hf],
                             add=True)

        def drain_s(hf):
            pltpu.make_async_copy(rows[hf], S_sh.at[dstb[0].at[0]],
                                  ssem[hf]).wait()

        def it(g, j, do_drain=True, do_idx=True, do_next=True):
            hf = j % 2
            h2 = 1 - hf
            if do_drain:
                drain_s(h2)
            if do_idx:
                fire_idx(g + 3, (j + 3) % 4)
            if do_next:
                fire_g(g + 1, h2, (j + 1) % 4)
            drain_g(hf)
            fire_s(hf, j)

        fire_idx(0, 0)
        fire_idx(1, 1)
        fire_idx(2, 2)
        fire_g(0, 0, 0)
        it(0, 0, do_drain=False)
        it(1, 1)
        it(2, 2)
        it(3, 3)

        def body(u, carry):
            it(4 * u + 0, 0)
            it(4 * u + 1, 1)
            it(4 * u + 2, 2)
            it(4 * u + 3, 3)
            return carry

        lax.fori_loop(1, GS // 4 - 1, body, 0)

        it(GS - 4, 0)
        it(GS - 3, 1, do_idx=False)
        it(GS - 2, 2, do_idx=False)
        it(GS - 1, 3, do_idx=False, do_next=False)
        drain_s(1)

        plsc.subcore_barrier()
        pltpu.sync_copy(S_sh.at[pl.ds(r0, RPT)], outS.at[c, pl.ds(r0, RPT)])

    return k(h, src3, dst3, z128)


_BLK = 1000


def _dense_body(Sp_ref, EAp_ref, x_ref, WeT_ref, be_ref,
                WlT_ref, bl_ref, WrT_ref, o_ref):
    s = Sp_ref[0] + Sp_ref[1]
    eac = EAp_ref[0] + EAp_ref[1]
    ea = eac[:, 0:ED]
    cnt = eac[:, ED:ED + 1]
    inv = 1.0 / jnp.maximum(cnt, 1.0)
    agg = (s + jnp.dot(ea, WeT_ref[...], preferred_element_type=jnp.float32)
           + cnt * be_ref[...]) * inv
    o_ref[...] = jnp.maximum(
        jnp.dot(agg, WlT_ref[...], preferred_element_type=jnp.float32)
        + bl_ref[...]
        + jnp.dot(x_ref[...], WrT_ref[...], preferred_element_type=jnp.float32),
        0.0)


def _dense(Sp, EAp, xin, WeT, be, WlT, bl, WrT):
    return pl.pallas_call(
        _dense_body,
        grid=(N // _BLK,),
        in_specs=[
            pl.BlockSpec((NC, _BLK, D), lambda i: (0, i, 0)),
            pl.BlockSpec((NC, _BLK, D), lambda i: (0, i, 0)),
            pl.BlockSpec((_BLK, D), lambda i: (i, 0)),
            pl.BlockSpec((ED, D), lambda i: (0, 0)),
            pl.BlockSpec((1, D), lambda i: (0, 0)),
            pl.BlockSpec((D, D), lambda i: (0, 0)),
            pl.BlockSpec((1, D), lambda i: (0, 0)),
            pl.BlockSpec((D, D), lambda i: (0, 0)),
        ],
        out_specs=pl.BlockSpec((_BLK, D), lambda i: (i, 0)),
        out_shape=jax.ShapeDtypeStruct((N, D), jnp.float32),
    )(Sp, EAp, xin, WeT, be, WlT, bl, WrT)


def kernel(x, edge_index, edge_attr, Wl0, bl0, Wr0, We0, be0,
           Wl1, bl1, Wr1, We1, be1):
    src = edge_index[0]
    dst = edge_index[1]
    z128 = jnp.zeros((RPT, D), jnp.float32)
    init128 = jnp.zeros((C, D), jnp.float32).at[:, ED].set(1.0)

    src3 = src.reshape(E // CS, 1, CS)
    dst3 = dst.reshape(E // CS, 1, CS)
    dst3e = dst.reshape(E // C, 1, C)

    EAp = _edge_agg(dst3e, edge_attr, z128, init128)
    S0p = _spmm(x, src3, dst3, z128)
    h = _dense(S0p, EAp, x, We0.T, be0.reshape(1, D), Wl0.T,
               bl0.reshape(1, D), Wr0.T)
    S1p = _spmm(h, src3, dst3, z128)
    out = _dense(S1p, EAp, h, We1.T, be1.reshape(1, D), Wl1.T,
                 bl1.reshape(1, D), Wr1.T)
    return out
